# baseline (device time: 1486119 ns/iter reference)
import jax
import jax.numpy as jnp
from jax import lax
from jax.experimental import pallas as pl
from jax.experimental.pallas import tpu as pltpu

N_DEV = 32


def kernel(x, w_mat, scale_x, scale_w):
    m, k = x.shape
    _, n = w_mat.shape
    m_per = m // N_DEV

    def body(x_ref, w_ref, sx_ref, sw_ref, out_ref,
             comm_ref, send_sems, recv_sems):
        my = lax.axis_index("i")
        left = lax.rem(my - 1 + N_DEV, N_DEV)
        right = lax.rem(my + 1, N_DEV)

        barrier_sem = pltpu.get_barrier_semaphore()
        for nbr in (left, right):
            pl.semaphore_signal(
                barrier_sem, inc=1,
                device_id=(nbr,), device_id_type=pl.DeviceIdType.MESH,
            )
        pl.semaphore_wait(barrier_sem, 2)

        wb = w_ref[...].astype(jnp.bfloat16)

        def partial(c):
            xc = x_ref[pl.ds(c * m_per, m_per), :].astype(jnp.bfloat16)
            return jnp.dot(xc, wb, preferred_element_type=jnp.float32)

        comm_ref[0] = partial(lax.rem(my - 1 + N_DEV, N_DEV))

        for h in range(N_DEV - 1):
            send_slot = h % 2
            recv_slot = (h + 1) % 2
            rdma = pltpu.make_async_remote_copy(
                src_ref=comm_ref.at[send_slot],
                dst_ref=comm_ref.at[recv_slot],
                send_sem=send_sems.at[send_slot],
                recv_sem=recv_sems.at[recv_slot],
                device_id=(right,),
                device_id_type=pl.DeviceIdType.MESH,
            )
            rdma.start()
            rdma.wait()
            c = lax.rem(my - 2 - h + 2 * N_DEV, N_DEV)
            comm_ref[recv_slot] = comm_ref[recv_slot] + partial(c)

        acc = comm_ref[(N_DEV - 1) % 2]
        y = acc * (sx_ref[0, 0] * sw_ref[0, 0])
        z = jnp.clip(y, -60.0, 60.0)
        out_ref[...] = y * (1.0 / (1.0 + jnp.exp(-z)))

    return pl.pallas_call(
        body,
        out_shape=jax.ShapeDtypeStruct((m_per, n), jnp.float32),
        in_specs=[
            pl.BlockSpec(memory_space=pltpu.VMEM),
            pl.BlockSpec(memory_space=pltpu.VMEM),
            pl.BlockSpec(memory_space=pltpu.SMEM),
            pl.BlockSpec(memory_space=pltpu.SMEM),
        ],
        out_specs=pl.BlockSpec(memory_space=pltpu.VMEM),
        scratch_shapes=[
            pltpu.VMEM((2, m_per, n), jnp.float32),
            pltpu.SemaphoreType.DMA((2,)),
            pltpu.SemaphoreType.DMA((2,)),
        ],
        compiler_params=pltpu.CompilerParams(collective_id=0),
    )(x, w_mat, scale_x.reshape(1, 1), scale_w.reshape(1, 1))


# device time: 441167 ns/iter; 3.3686x vs baseline; 3.3686x over previous
import jax
import jax.numpy as jnp
from jax import lax
from jax.experimental import pallas as pl
from jax.experimental.pallas import tpu as pltpu

N_DEV = 32

CYCLE = (0, 3, 4, 7, 15, 12, 11, 8, 16, 19, 20, 23, 31, 28, 27, 24,
         25, 26, 29, 30, 22, 21, 18, 17, 9, 10, 13, 14, 6, 5, 2, 1)
INV = tuple({d: p for p, d in enumerate(CYCLE)}[i] for i in range(N_DEV))


def kernel(x, w_mat, scale_x, scale_w):
    m, k = x.shape
    _, n = w_mat.shape
    m_per = m // N_DEV
    nh = n // 2

    my = lax.axis_index("i")
    cyc = jnp.asarray(CYCLE, jnp.int32)
    pos = jnp.asarray(INV, jnp.int32)[my]
    h = jnp.arange(N_DEV - 1, dtype=jnp.int32)
    params = jnp.concatenate([
        jnp.stack([cyc[(pos + 1) % N_DEV],
                   cyc[(pos - 1) % N_DEV]]),
        cyc[(pos - 2 - h) % N_DEV],
        cyc[(pos + 2 + h) % N_DEV],
    ]).astype(jnp.int32)

    def body(params_ref, x_ref, w_ref, sx_ref, sw_ref, out_ref,
             comm_f, comm_b, send_f, recv_f, send_b, recv_b):
        nxt = params_ref[0]
        prv = params_ref[1]

        barrier_sem = pltpu.get_barrier_semaphore()
        for nbr in (nxt, prv):
            pl.semaphore_signal(
                barrier_sem, inc=1,
                device_id=(nbr,), device_id_type=pl.DeviceIdType.MESH,
            )
        pl.semaphore_wait(barrier_sem, 2)

        wf = w_ref[:, :nh].astype(jnp.bfloat16)
        wb = w_ref[:, nh:].astype(jnp.bfloat16)

        def partial(c, whalf):
            xc = x_ref[pl.ds(c * m_per, m_per), :].astype(jnp.bfloat16)
            return jnp.dot(xc, whalf, preferred_element_type=jnp.float32)

        comm_f[0] = partial(prv, wf).astype(jnp.bfloat16)
        comm_b[0] = partial(nxt, wb).astype(jnp.bfloat16)

        scale = sx_ref[0, 0] * sw_ref[0, 0]

        for hop in range(N_DEV - 1):
            s = hop % 2
            r = (hop + 1) % 2
            last = hop == N_DEV - 2

            rdma_f = pltpu.make_async_remote_copy(
                src_ref=comm_f.at[s], dst_ref=comm_f.at[r],
                send_sem=send_f.at[s], recv_sem=recv_f.at[r],
                device_id=(nxt,), device_id_type=pl.DeviceIdType.MESH,
            )
            rdma_f.start()
            rdma_b = pltpu.make_async_remote_copy(
                src_ref=comm_b.at[s], dst_ref=comm_b.at[r],
                send_sem=send_b.at[s], recv_sem=recv_b.at[r],
                device_id=(prv,), device_id_type=pl.DeviceIdType.MESH,
            )
            rdma_b.start()

            p_f = partial(params_ref[2 + hop], wf)
            p_b = partial(params_ref[2 + (N_DEV - 1) + hop], wb)

            rdma_f.wait()
            acc_f = comm_f[r].astype(jnp.float32) + p_f
            if last:
                y = acc_f * scale
                z = jnp.clip(y, -60.0, 60.0)
                out_ref[:, :nh] = y * (1.0 / (1.0 + jnp.exp(-z)))
            else:
                comm_f[r] = acc_f.astype(jnp.bfloat16)

            rdma_b.wait()
            acc_b = comm_b[r].astype(jnp.float32) + p_b
            if last:
                y = acc_b * scale
                z = jnp.clip(y, -60.0, 60.0)
                out_ref[:, nh:] = y * (1.0 / (1.0 + jnp.exp(-z)))
            else:
                comm_b[r] = acc_b.astype(jnp.bfloat16)

    return pl.pallas_call(
        body,
        out_shape=jax.ShapeDtypeStruct((m_per, n), jnp.float32),
        in_specs=[
            pl.BlockSpec(memory_space=pltpu.SMEM),
            pl.BlockSpec(memory_space=pltpu.VMEM),
            pl.BlockSpec(memory_space=pltpu.VMEM),
            pl.BlockSpec(memory_space=pltpu.SMEM),
            pl.BlockSpec(memory_space=pltpu.SMEM),
        ],
        out_specs=pl.BlockSpec(memory_space=pltpu.VMEM),
        scratch_shapes=[
            pltpu.VMEM((2, m_per, nh), jnp.bfloat16),
            pltpu.VMEM((2, m_per, nh), jnp.bfloat16),
            pltpu.SemaphoreType.DMA((2,)),
            pltpu.SemaphoreType.DMA((2,)),
            pltpu.SemaphoreType.DMA((2,)),
            pltpu.SemaphoreType.DMA((2,)),
        ],
        compiler_params=pltpu.CompilerParams(collective_id=0),
    )(params, x, w_mat, scale_x.reshape(1, 1), scale_w.reshape(1, 1))


# device time: 367411 ns/iter; 4.0448x vs baseline; 1.2007x over previous
import jax
import jax.numpy as jnp
from jax import lax
from jax.experimental import pallas as pl
from jax.experimental.pallas import tpu as pltpu

N_DEV = 32

CYCLE = (0, 3, 4, 7, 15, 12, 11, 8, 16, 19, 20, 23, 31, 28, 27, 24,
         25, 26, 29, 30, 22, 21, 18, 17, 9, 10, 13, 14, 6, 5, 2, 1)
INV = tuple({d: p for p, d in enumerate(CYCLE)}[i] for i in range(N_DEV))


def kernel(x, w_mat, scale_x, scale_w):
    m, k = x.shape
    _, n = w_mat.shape
    m_per = m // N_DEV
    nq = n // 4

    my = lax.axis_index("i")
    cyc = jnp.asarray(CYCLE, jnp.int32)
    pos = jnp.asarray(INV, jnp.int32)[my]
    h = jnp.arange(N_DEV - 1, dtype=jnp.int32)
    params = jnp.concatenate([
        jnp.stack([cyc[(pos + 1) % N_DEV],
                   cyc[(pos - 1) % N_DEV]]),
        cyc[(pos - 2 - h) % N_DEV],
        cyc[(pos + 2 + h) % N_DEV],
    ]).astype(jnp.int32)

    def body(params_ref, x_ref, w_ref, sx_ref, sw_ref, out_ref,
             comm_f1, comm_f2, comm_b1, comm_b2,
             send_f1, recv_f1, send_f2, recv_f2,
             send_b1, recv_b1, send_b2, recv_b2):
        nxt = params_ref[0]
        prv = params_ref[1]

        barrier_sem = pltpu.get_barrier_semaphore()
        for nbr in (nxt, prv):
            pl.semaphore_signal(
                barrier_sem, inc=1,
                device_id=(nbr,), device_id_type=pl.DeviceIdType.MESH,
            )
        pl.semaphore_wait(barrier_sem, 2)

        scale = sx_ref[0, 0] * sw_ref[0, 0]

        rings = (
            (comm_f1, send_f1, recv_f1, nxt, 2, 0 * nq),
            (comm_b1, send_b1, recv_b1, prv, 2 + (N_DEV - 1), 2 * nq),
            (comm_f2, send_f2, recv_f2, nxt, 2, 1 * nq),
            (comm_b2, send_b2, recv_b2, prv, 2 + (N_DEV - 1), 3 * nq),
        )
        out_col = (0 * nq, 2 * nq, 1 * nq, 3 * nq)

        wq = [w_ref[:, c * nq:(c + 1) * nq].astype(jnp.bfloat16)
              for c in range(4)]

        def partial(c, ring_idx):
            xc = x_ref[pl.ds(c * m_per, m_per), :].astype(jnp.bfloat16)
            w_idx = out_col[ring_idx] // nq
            return jnp.dot(xc, wq[w_idx], preferred_element_type=jnp.float32)

        def make(ring, hop):
            comm, ssem, rsem, tgt, _, _ = ring
            s = hop % 2
            r = (hop + 1) % 2
            return pltpu.make_async_remote_copy(
                src_ref=comm.at[s], dst_ref=comm.at[r],
                send_sem=ssem.at[s], recv_sem=rsem.at[r],
                device_id=(tgt,), device_id_type=pl.DeviceIdType.MESH,
            )

        for i, ring in enumerate(rings):
            seed = prv if ring[3] is nxt else nxt
            ring[0][0] = partial(seed, i).astype(jnp.bfloat16)

        inflight = [None] * 4
        pending = [None] * 4
        for i in (0, 1):
            inflight[i] = make(rings[i], 0)
            inflight[i].start()
        inflight[0].wait_send()
        for i in (2, 3):
            inflight[i] = make(rings[i], 0)
            inflight[i].start()

        for hop in range(N_DEV - 1):
            r = (hop + 1) % 2
            last = hop == N_DEV - 2
            for i, ring in enumerate(rings):
                comm, _, _, _, cbase, _ = ring
                p = partial(params_ref[cbase + hop], i)
                inflight[i].wait_recv()
                if pending[i] is not None:
                    pending[i].wait_send()
                acc = comm[r].astype(jnp.float32) + p
                if last:
                    y = acc * scale
                    z = jnp.clip(y, -60.0, 60.0)
                    c0 = out_col[i]
                    out_ref[:, c0:c0 + nq] = y * (1.0 / (1.0 + jnp.exp(-z)))
                else:
                    comm[r] = acc.astype(jnp.bfloat16)
                    nxt_rdma = make(ring, hop + 1)
                    nxt_rdma.start()
                    pending[i] = None if (hop == 0 and i == 0) else inflight[i]
                    inflight[i] = nxt_rdma

        for i in range(4):
            inflight[i].wait_send()

    return pl.pallas_call(
        body,
        out_shape=jax.ShapeDtypeStruct((m_per, n), jnp.float32),
        in_specs=[
            pl.BlockSpec(memory_space=pltpu.SMEM),
            pl.BlockSpec(memory_space=pltpu.VMEM),
            pl.BlockSpec(memory_space=pltpu.VMEM),
            pl.BlockSpec(memory_space=pltpu.SMEM),
            pl.BlockSpec(memory_space=pltpu.SMEM),
        ],
        out_specs=pl.BlockSpec(memory_space=pltpu.VMEM),
        scratch_shapes=[
            pltpu.VMEM((2, m_per, nq), jnp.bfloat16),
            pltpu.VMEM((2, m_per, nq), jnp.bfloat16),
            pltpu.VMEM((2, m_per, nq), jnp.bfloat16),
            pltpu.VMEM((2, m_per, nq), jnp.bfloat16),
            pltpu.SemaphoreType.DMA((2,)),
            pltpu.SemaphoreType.DMA((2,)),
            pltpu.SemaphoreType.DMA((2,)),
            pltpu.SemaphoreType.DMA((2,)),
            pltpu.SemaphoreType.DMA((2,)),
            pltpu.SemaphoreType.DMA((2,)),
            pltpu.SemaphoreType.DMA((2,)),
            pltpu.SemaphoreType.DMA((2,)),
        ],
        compiler_params=pltpu.CompilerParams(collective_id=0),
    )(params, x, w_mat, scale_x.reshape(1, 1), scale_w.reshape(1, 1))
